# baseline (speedup 1.0000x reference)
import functools

import jax
import jax.numpy as jnp
from jax import lax
from jax.experimental import pallas as pl
from jax.experimental.pallas import tpu as pltpu
from jax.experimental.pallas import tpu_sc as plsc

_D = 32
_L = 20
_ROW = _L * _D
_LANES = 16
_NC = 2
_NS = 16
_NW = _NC * _NS
_CH = 64
_LPAD = 24


@functools.lru_cache(maxsize=None)
def _make_sc_call(BN: int):
    assert BN % (_NW * 2 * _CH) == 0, BN
    rows_per_w = BN // _NW
    npairs = rows_per_w // (2 * _CH)
    mesh = plsc.VectorSubcoreMesh(core_axis_name="c", subcore_axis_name="s")

    def body(xf_hbm, mt_hbm, out_hbm,
             mbuf, inbuf0, inbuf1, outbuf, msem, sem0, sem1, osem):
        c = lax.axis_index("c")
        s = lax.axis_index("s")
        wid = s * _NC + c
        base = wid * rows_per_w

        pltpu.async_copy(mt_hbm.at[pl.ds(0, _LPAD)], mbuf, msem).wait()
        mreg = [mbuf[l, pl.ds(h * _LANES, _LANES)]
                for l in range(_L) for h in range(2)]

        def compute(inbuf, cbase):
            def row_body(r, carry):
                acc0 = inbuf[r, pl.ds(0, _LANES)] * mreg[0]
                acc1 = inbuf[r, pl.ds(_LANES, _LANES)] * mreg[1]
                for l in range(1, _L):
                    acc0 = acc0 + inbuf[r, pl.ds(l * _D, _LANES)] * mreg[2 * l]
                    acc1 = acc1 + inbuf[r, pl.ds(l * _D + _LANES, _LANES)] * mreg[2 * l + 1]
                outbuf[r, pl.ds(0, _LANES)] = acc0
                outbuf[r, pl.ds(_LANES, _LANES)] = acc1
                return carry
            lax.fori_loop(0, _CH, row_body, 0, unroll=2)
            pltpu.async_copy(outbuf, out_hbm.at[pl.ds(cbase, _CH)], osem).wait()

        pltpu.async_copy(xf_hbm.at[pl.ds(base, _CH)], inbuf0, sem0)

        def step(k, carry):
            c0 = base + (2 * k) * _CH
            pltpu.make_async_copy(xf_hbm.at[pl.ds(c0, _CH)], inbuf0, sem0).wait()
            pltpu.async_copy(xf_hbm.at[pl.ds(c0 + _CH, _CH)], inbuf1, sem1)
            compute(inbuf0, c0)
            pltpu.make_async_copy(
                xf_hbm.at[pl.ds(c0 + _CH, _CH)], inbuf1, sem1).wait()

            @pl.when(k < npairs - 1)
            def _():
                pltpu.async_copy(
                    xf_hbm.at[pl.ds(c0 + 2 * _CH, _CH)], inbuf0, sem0)

            compute(inbuf1, c0 + _CH)
            return carry

        lax.fori_loop(0, npairs, step, 0)

    return pl.kernel(
        body,
        out_type=jax.ShapeDtypeStruct((BN, _D), jnp.float32),
        mesh=mesh,
        scratch_types=[
            pltpu.VMEM((_LPAD, _D), jnp.float32),
            pltpu.VMEM((_CH, _ROW), jnp.float32),
            pltpu.VMEM((_CH, _ROW), jnp.float32),
            pltpu.VMEM((_CH, _D), jnp.float32),
            pltpu.SemaphoreType.DMA,
            pltpu.SemaphoreType.DMA,
            pltpu.SemaphoreType.DMA,
            pltpu.SemaphoreType.DMA,
        ],
    )


def kernel(x, mask_table):
    B, N, L, D = x.shape
    assert (L, D) == (_L, _D)
    BN = B * N
    xf = x.reshape(BN, L * D)
    out = _make_sc_call(BN)(xf, mask_table)
    return out.reshape(B, N, D)


# speedup vs baseline: 1.0007x; 1.0007x over previous
import functools

import jax
import jax.numpy as jnp
from jax import lax
from jax.experimental import pallas as pl
from jax.experimental.pallas import tpu as pltpu
from jax.experimental.pallas import tpu_sc as plsc

_D = 32
_L = 20
_ROW = _L * _D
_LANES = 16
_NC = 2
_NS = 16
_NW = _NC * _NS
_CH = 64
_LPAD = 24


@functools.lru_cache(maxsize=None)
def _make_sc_call(BN: int):
    assert BN % (_NW * 2 * _CH) == 0, BN
    rows_per_w = BN // _NW
    npairs = rows_per_w // (2 * _CH)
    mesh = plsc.VectorSubcoreMesh(core_axis_name="c", subcore_axis_name="s")

    def body(xf_hbm, mt_hbm, out_hbm,
             mbuf, inbuf0, inbuf1, outbuf, msem, sem0, sem1, osem):
        c = lax.axis_index("c")
        s = lax.axis_index("s")
        wid = s * _NC + c
        base = wid * rows_per_w

        pltpu.async_copy(mt_hbm.at[pl.ds(0, _LPAD)], mbuf, msem).wait()
        mreg = [mbuf[l, pl.ds(h * _LANES, _LANES)]
                for l in range(_L) for h in range(2)]

        def _tree(vs):
            while len(vs) > 1:
                nxt = [vs[i] + vs[i + 1] for i in range(0, len(vs) - 1, 2)]
                if len(vs) % 2:
                    nxt.append(vs[-1])
                vs = nxt
            return vs[0]

        def compute(inbuf, cbase):
            @plsc.parallel_loop(0, _CH, 1, unroll=4)
            def _rows(r):
                p = [inbuf[r, pl.ds(j * _LANES, _LANES)] * mreg[j]
                     for j in range(2 * _L)]
                outbuf[r, pl.ds(0, _LANES)] = _tree(p[0::2])
                outbuf[r, pl.ds(_LANES, _LANES)] = _tree(p[1::2])

            pltpu.async_copy(outbuf, out_hbm.at[pl.ds(cbase, _CH)], osem).wait()

        pltpu.async_copy(xf_hbm.at[pl.ds(base, _CH)], inbuf0, sem0)

        def step(k, carry):
            c0 = base + (2 * k) * _CH
            pltpu.make_async_copy(xf_hbm.at[pl.ds(c0, _CH)], inbuf0, sem0).wait()
            pltpu.async_copy(xf_hbm.at[pl.ds(c0 + _CH, _CH)], inbuf1, sem1)
            compute(inbuf0, c0)
            pltpu.make_async_copy(
                xf_hbm.at[pl.ds(c0 + _CH, _CH)], inbuf1, sem1).wait()

            @pl.when(k < npairs - 1)
            def _():
                pltpu.async_copy(
                    xf_hbm.at[pl.ds(c0 + 2 * _CH, _CH)], inbuf0, sem0)

            compute(inbuf1, c0 + _CH)
            return carry

        lax.fori_loop(0, npairs, step, 0)

    return pl.kernel(
        body,
        out_type=jax.ShapeDtypeStruct((BN, _D), jnp.float32),
        mesh=mesh,
        scratch_types=[
            pltpu.VMEM((_LPAD, _D), jnp.float32),
            pltpu.VMEM((_CH, _ROW), jnp.float32),
            pltpu.VMEM((_CH, _ROW), jnp.float32),
            pltpu.VMEM((_CH, _D), jnp.float32),
            pltpu.SemaphoreType.DMA,
            pltpu.SemaphoreType.DMA,
            pltpu.SemaphoreType.DMA,
            pltpu.SemaphoreType.DMA,
        ],
    )


def kernel(x, mask_table):
    B, N, L, D = x.shape
    assert (L, D) == (_L, _D)
    BN = B * N
    xf = x.reshape(BN, L * D)
    out = _make_sc_call(BN)(xf, mask_table)
    return out.reshape(B, N, D)


# speedup vs baseline: 1.0244x; 1.0236x over previous
import functools

import jax
import jax.numpy as jnp
from jax import lax
from jax.experimental import pallas as pl
from jax.experimental.pallas import tpu as pltpu
from jax.experimental.pallas import tpu_sc as plsc

_D = 32
_L = 20
_ROW = _L * _D
_LANES = 16
_NC = 2
_NS = 16
_NW = _NC * _NS
_CH = 64
_LPAD = 24
_RT = 8
_GL = 10


def _tree(vs):
    while len(vs) > 1:
        nxt = [vs[i] + vs[i + 1] for i in range(0, len(vs) - 1, 2)]
        if len(vs) % 2:
            nxt.append(vs[-1])
        vs = nxt
    return vs[0]


@functools.lru_cache(maxsize=None)
def _make_sc_call(BN: int):
    assert BN % (_NW * 2 * _CH) == 0, BN
    rows_per_w = BN // _NW
    npairs = rows_per_w // (2 * _CH)
    in_sz = _CH * _ROW
    out_sz = _CH * _D
    mesh = plsc.VectorSubcoreMesh(core_axis_name="c", subcore_axis_name="s")

    def body(xf_hbm, mt_hbm, out_hbm,
             mbuf, inbuf0, inbuf1, outbuf0, outbuf1,
             msem, sem0, sem1, osem0, osem1):
        c = lax.axis_index("c")
        s = lax.axis_index("s")
        wid = s * _NC + c
        base = wid * rows_per_w

        pltpu.async_copy(mt_hbm.at[pl.ds(0, _LPAD * _D)], mbuf, msem).wait()

        def compute(inbuf, outbuf, osem, cbase, k):
            @pl.when(k > 0)
            def _():
                pltpu.make_async_copy(
                    outbuf, out_hbm.at[pl.ds(base * _D, out_sz)], osem).wait()

            def tile_body(t, carry):
                tb = t * _RT * _ROW
                ob = t * _RT * _D
                for g in range(2):
                    mreg = [mbuf[pl.ds((g * _GL * 2 + j) * _LANES, _LANES)]
                            for j in range(2 * _GL)]
                    for r in range(_RT):
                        rb = tb + r * _ROW + g * _GL * _D
                        p = [inbuf[pl.ds(rb + j * _LANES, _LANES)] * mreg[j]
                             for j in range(2 * _GL)]
                        e = _tree(p[0::2])
                        o = _tree(p[1::2])
                        if g == 0:
                            outbuf[pl.ds(ob + r * _D, _LANES)] = e
                            outbuf[pl.ds(ob + r * _D + _LANES, _LANES)] = o
                        else:
                            plsc.addupdate(
                                outbuf.at[pl.ds(ob + r * _D, _LANES)], e)
                            plsc.addupdate(
                                outbuf.at[pl.ds(ob + r * _D + _LANES, _LANES)], o)
                return carry

            lax.fori_loop(0, _CH // _RT, tile_body, 0)
            pltpu.async_copy(
                outbuf, out_hbm.at[pl.ds(cbase * _D, out_sz)], osem)

        pltpu.async_copy(
            xf_hbm.at[pl.ds(base * _ROW, in_sz)], inbuf0, sem0)

        def step(k, carry):
            c0 = base + (2 * k) * _CH
            pltpu.make_async_copy(
                xf_hbm.at[pl.ds(c0 * _ROW, in_sz)], inbuf0, sem0).wait()
            pltpu.async_copy(
                xf_hbm.at[pl.ds((c0 + _CH) * _ROW, in_sz)], inbuf1, sem1)
            compute(inbuf0, outbuf0, osem0, c0, k)
            pltpu.make_async_copy(
                xf_hbm.at[pl.ds((c0 + _CH) * _ROW, in_sz)], inbuf1, sem1).wait()

            @pl.when(k < npairs - 1)
            def _():
                pltpu.async_copy(
                    xf_hbm.at[pl.ds((c0 + 2 * _CH) * _ROW, in_sz)],
                    inbuf0, sem0)

            compute(inbuf1, outbuf1, osem1, c0 + _CH, k)
            return carry

        lax.fori_loop(0, npairs, step, 0)
        pltpu.make_async_copy(
            outbuf0, out_hbm.at[pl.ds(base * _D, out_sz)], osem0).wait()
        pltpu.make_async_copy(
            outbuf1, out_hbm.at[pl.ds(base * _D, out_sz)], osem1).wait()

    return pl.kernel(
        body,
        out_type=jax.ShapeDtypeStruct((BN * _D,), jnp.float32),
        mesh=mesh,
        scratch_types=[
            pltpu.VMEM((_LPAD * _D,), jnp.float32),
            pltpu.VMEM((in_sz,), jnp.float32),
            pltpu.VMEM((in_sz,), jnp.float32),
            pltpu.VMEM((out_sz,), jnp.float32),
            pltpu.VMEM((out_sz,), jnp.float32),
            pltpu.SemaphoreType.DMA,
            pltpu.SemaphoreType.DMA,
            pltpu.SemaphoreType.DMA,
            pltpu.SemaphoreType.DMA,
            pltpu.SemaphoreType.DMA,
        ],
    )


def kernel(x, mask_table):
    B, N, L, D = x.shape
    assert (L, D) == (_L, _D)
    BN = B * N
    xf = x.reshape(BN * L * D)
    mt = mask_table.reshape(-1)
    out = _make_sc_call(BN)(xf, mt)
    return out.reshape(B, N, D)


# speedup vs baseline: 1.0249x; 1.0005x over previous
import functools

import jax
import jax.numpy as jnp
from jax import lax
from jax.experimental import pallas as pl
from jax.experimental.pallas import tpu as pltpu
from jax.experimental.pallas import tpu_sc as plsc

_D = 32
_L = 20
_ROW = _L * _D
_LANES = 16
_NC = 2
_NS = 16
_NW = _NC * _NS
_CH = 64
_LPAD = 24
_RT = 8
_GL = 10
_NSPLIT = 4


def _tree(vs):
    while len(vs) > 1:
        nxt = [vs[i] + vs[i + 1] for i in range(0, len(vs) - 1, 2)]
        if len(vs) % 2:
            nxt.append(vs[-1])
        vs = nxt
    return vs[0]


@functools.lru_cache(maxsize=None)
def _make_sc_call(BN: int):
    assert BN % (_NW * 2 * _CH) == 0, BN
    rows_per_w = BN // _NW
    npairs = rows_per_w // (2 * _CH)
    in_sz = _CH * _ROW
    out_sz = _CH * _D
    mesh = plsc.VectorSubcoreMesh(core_axis_name="c", subcore_axis_name="s")

    def body(xf_hbm, mt_hbm, out_hbm,
             mbuf, inbuf0, inbuf1, outbuf0, outbuf1,
             msem, sem0, sem1, osem0, osem1):
        c = lax.axis_index("c")
        s = lax.axis_index("s")
        wid = s * _NC + c
        base = wid * rows_per_w

        pltpu.async_copy(mt_hbm.at[pl.ds(0, _LPAD * _D)], mbuf, msem).wait()

        def compute(inbuf, outbuf, osem, cbase, k):
            @pl.when(k > 0)
            def _():
                pltpu.make_async_copy(
                    outbuf, out_hbm.at[pl.ds(base * _D, out_sz)], osem).wait()

            def tile_body(t, carry):
                tb = t * _RT * _ROW
                ob = t * _RT * _D
                for g in range(2):
                    mreg = [mbuf[pl.ds((g * _GL * 2 + j) * _LANES, _LANES)]
                            for j in range(2 * _GL)]
                    for r in range(_RT):
                        rb = tb + r * _ROW + g * _GL * _D
                        p = [inbuf[pl.ds(rb + j * _LANES, _LANES)] * mreg[j]
                             for j in range(2 * _GL)]
                        e = _tree(p[0::2])
                        o = _tree(p[1::2])
                        if g == 0:
                            outbuf[pl.ds(ob + r * _D, _LANES)] = e
                            outbuf[pl.ds(ob + r * _D + _LANES, _LANES)] = o
                        else:
                            plsc.addupdate(
                                outbuf.at[pl.ds(ob + r * _D, _LANES)], e)
                            plsc.addupdate(
                                outbuf.at[pl.ds(ob + r * _D + _LANES, _LANES)], o)
                return carry

            lax.fori_loop(0, _CH // _RT, tile_body, 0)
            pltpu.async_copy(
                outbuf, out_hbm.at[pl.ds(cbase * _D, out_sz)], osem)

        sub = in_sz // _NSPLIT

        def start_in(row0, inbuf, sem):
            for i in range(_NSPLIT):
                pltpu.async_copy(
                    xf_hbm.at[pl.ds(row0 * _ROW + i * sub, sub)],
                    inbuf.at[pl.ds(i * sub, sub)], sem)

        def wait_in(row0, inbuf, sem):
            pltpu.make_async_copy(
                xf_hbm.at[pl.ds(row0 * _ROW, in_sz)], inbuf, sem).wait()

        start_in(base, inbuf0, sem0)

        def step(k, carry):
            c0 = base + (2 * k) * _CH
            wait_in(c0, inbuf0, sem0)
            start_in(c0 + _CH, inbuf1, sem1)
            compute(inbuf0, outbuf0, osem0, c0, k)
            wait_in(c0 + _CH, inbuf1, sem1)

            @pl.when(k < npairs - 1)
            def _():
                start_in(c0 + 2 * _CH, inbuf0, sem0)

            compute(inbuf1, outbuf1, osem1, c0 + _CH, k)
            return carry

        lax.fori_loop(0, npairs, step, 0)
        pltpu.make_async_copy(
            outbuf0, out_hbm.at[pl.ds(base * _D, out_sz)], osem0).wait()
        pltpu.make_async_copy(
            outbuf1, out_hbm.at[pl.ds(base * _D, out_sz)], osem1).wait()

    return pl.kernel(
        body,
        out_type=jax.ShapeDtypeStruct((BN * _D,), jnp.float32),
        mesh=mesh,
        scratch_types=[
            pltpu.VMEM((_LPAD * _D,), jnp.float32),
            pltpu.VMEM((in_sz,), jnp.float32),
            pltpu.VMEM((in_sz,), jnp.float32),
            pltpu.VMEM((out_sz,), jnp.float32),
            pltpu.VMEM((out_sz,), jnp.float32),
            pltpu.SemaphoreType.DMA,
            pltpu.SemaphoreType.DMA,
            pltpu.SemaphoreType.DMA,
            pltpu.SemaphoreType.DMA,
            pltpu.SemaphoreType.DMA,
        ],
    )


def kernel(x, mask_table):
    B, N, L, D = x.shape
    assert (L, D) == (_L, _D)
    BN = B * N
    xf = x.reshape(BN * L * D)
    mt = mask_table.reshape(-1)
    out = _make_sc_call(BN)(xf, mt)
    return out.reshape(B, N, D)


# speedup vs baseline: 6.3983x; 6.2428x over previous
import functools

import jax
import jax.numpy as jnp
from jax import lax
from jax.experimental import pallas as pl
from jax.experimental.pallas import tpu as pltpu
from jax.experimental.pallas import tpu_sc as plsc

_D = 32
_L = 20
_B = 4096
_N = 26
_LANES = 16
_NC = 2
_NS = 16
_NW = _NC * _NS
_BSLAB = _B // _NW
_DH = 8
_NQ = _D // _DH
_VPB = _BSLAB // _LANES


def _tree(vs):
    while len(vs) > 1:
        nxt = [vs[i] + vs[i + 1] for i in range(0, len(vs) - 1, 2)]
        if len(vs) % 2:
            nxt.append(vs[-1])
        vs = nxt
    return vs[0]


@functools.lru_cache(maxsize=None)
def _make_sc_call():
    mesh = plsc.VectorSubcoreMesh(core_axis_name="c", subcore_axis_name="s")

    def body(xt_hbm, mt_hbm, out_hbm,
             mbuf, inbuf0, inbuf1, outbuf0, outbuf1,
             msem, sem0, sem1, osem0, osem1):
        c = lax.axis_index("c")
        s = lax.axis_index("s")
        wid = s * _NC + c
        b0 = pl.multiple_of(wid * _BSLAB, _BSLAB)

        pltpu.async_copy(mt_hbm, mbuf, msem).wait()

        def compute(inbuf, outbuf, osem, n, d0, first):
            @pl.when(jnp.logical_not(first))
            def _():
                pltpu.make_async_copy(
                    outbuf,
                    out_hbm.at[0, pl.ds(d0, _DH), pl.ds(b0, _BSLAB)],
                    osem).wait()

            def db_body(db, carry):
                mbase = (d0 + db) * _L
                mreg = [mbuf[mbase + l, :] for l in range(_L)]
                for vi in range(_VPB):
                    p = [inbuf[l, db, pl.ds(vi * _LANES, _LANES)] * mreg[l]
                         for l in range(_L)]
                    outbuf[db, pl.ds(vi * _LANES, _LANES)] = _tree(p)
                return carry

            lax.fori_loop(0, _DH, db_body, 0)
            pltpu.async_copy(
                outbuf,
                out_hbm.at[n, pl.ds(d0, _DH), pl.ds(b0, _BSLAB)], osem)

        def start_in(n, d0, inbuf, sem):
            pltpu.async_copy(
                xt_hbm.at[n, :, pl.ds(d0, _DH), pl.ds(b0, _BSLAB)],
                inbuf, sem)

        def wait_in(n, d0, inbuf, sem):
            pltpu.make_async_copy(
                xt_hbm.at[n, :, pl.ds(d0, _DH), pl.ds(b0, _BSLAB)],
                inbuf, sem).wait()

        start_in(0, 0, inbuf0, sem0)

        bufs = [(inbuf0, outbuf0, sem0, osem0), (inbuf1, outbuf1, sem1, osem1)]

        def step(n, carry):
            for q in range(_NQ):
                ib, ob, isem, osem = bufs[q % 2]
                nib, _, nisem, _ = bufs[(q + 1) % 2]
                wait_in(n, q * _DH, ib, isem)
                if q < _NQ - 1:
                    start_in(n, (q + 1) * _DH, nib, nisem)
                else:
                    @pl.when(n < _N - 1)
                    def _():
                        start_in(n + 1, 0, nib, nisem)
                compute(ib, ob, osem, n, q * _DH, jnp.logical_and(n == 0, q < 2))
            return carry

        lax.fori_loop(0, _N, step, 0)
        pltpu.make_async_copy(
            outbuf0, out_hbm.at[0, pl.ds(0, _DH), pl.ds(b0, _BSLAB)],
            osem0).wait()
        pltpu.make_async_copy(
            outbuf1, out_hbm.at[0, pl.ds(_DH, _DH), pl.ds(b0, _BSLAB)],
            osem1).wait()

    return pl.kernel(
        body,
        out_type=jax.ShapeDtypeStruct((_N, _D, _B), jnp.float32),
        mesh=mesh,
        scratch_types=[
            pltpu.VMEM((_D * _L, _LANES), jnp.float32),
            pltpu.VMEM((_L, _DH, _BSLAB), jnp.float32),
            pltpu.VMEM((_L, _DH, _BSLAB), jnp.float32),
            pltpu.VMEM((_DH, _BSLAB), jnp.float32),
            pltpu.VMEM((_DH, _BSLAB), jnp.float32),
            pltpu.SemaphoreType.DMA,
            pltpu.SemaphoreType.DMA,
            pltpu.SemaphoreType.DMA,
            pltpu.SemaphoreType.DMA,
            pltpu.SemaphoreType.DMA,
        ],
    )


def kernel(x, mask_table):
    B, N, L, D = x.shape
    assert (B, N, L, D) == (_B, _N, _L, _D)
    xt = jnp.transpose(x, (1, 2, 3, 0))
    mb = jnp.broadcast_to(
        mask_table[:_L].T[:, :, None], (_D, _L, _LANES)).reshape(
            _D * _L, _LANES)
    out = _make_sc_call()(xt, mb)
    return jnp.transpose(out, (2, 0, 1))
